# baseline (speedup 1.0000x reference)
import functools

import jax
import jax.numpy as jnp
from jax import lax
from jax.experimental import pallas as pl
from jax.experimental.pallas import tpu as pltpu
from jax.experimental.pallas import tpu_sc as plsc

NTOKEN = 1000000
EMB_DIM = 64

_info = plsc.get_sparse_core_info()
_NC, _NS = _info.num_cores, _info.num_subcores
_NW = _NC * _NS

_NB = 16384
_NJ = 20
_B = _NB * _NJ
_BPW = _B // _NW
_CB = 16
_C = _CB * _NJ
_NCHUNK = _BPW // _C
_NBUF = 2
_NG = _NCHUNK // _NBUF


def _make_kernel():
    mesh = plsc.VectorSubcoreMesh(core_axis_name="c", subcore_axis_name="s")

    @functools.partial(
        pl.kernel,
        mesh=mesh,
        out_type=jax.ShapeDtypeStruct((_NB, _NJ, EMB_DIM), jnp.float32),
        scratch_types=[
            pltpu.VMEM((_NBUF, _C), jnp.int32),
            pltpu.VMEM((_NBUF, _CB, _NJ, EMB_DIM), jnp.float32),
            pltpu.SemaphoreType.DMA((_NBUF,)),
        ],
        compiler_params=pltpu.CompilerParams(use_tc_tiling_on_sc=True),
    )
    def emb_kernel(table_hbm, idx_hbm, out_hbm, idx_v, rows_v, gsem):
        wid = lax.axis_index("s") * _NC + lax.axis_index("c")
        base = wid * (_BPW // _NJ)

        def fire(t, b):
            pltpu.sync_copy(idx_hbm.at[wid, t], idx_v.at[b])

            def row16(q, _):
                r = q * 16
                iv = idx_v[b, pl.ds(r, 16)]
                for u in range(16):
                    rr = r + u
                    pltpu.async_copy(table_hbm.at[pl.ds(iv[u], 1)],
                                     rows_v.at[b, rr // _NJ,
                                               pl.ds(rr % _NJ, 1)],
                                     gsem.at[b])
                return _

            lax.fori_loop(0, _C // 16, row16, None)

        for b in range(_NBUF):
            fire(b, b)

        def group(g, _):
            for b in range(_NBUF):
                t = g * _NBUF + b
                pltpu.make_async_copy(out_hbm.at[pl.ds(0, _CB)],
                                      rows_v.at[b], gsem.at[b]).wait()
                pltpu.sync_copy(rows_v.at[b],
                                out_hbm.at[pl.ds(base + t * _CB, _CB)])

                @pl.when(g < _NG - 1)
                def _refill():
                    fire(t + _NBUF, b)
            return _

        lax.fori_loop(0, _NG, group, None)

    return emb_kernel


_emb_kernel = _make_kernel()


@jax.jit
def kernel(x, table):
    idx = x.astype(jnp.int32).reshape(_NW, _NCHUNK, _C)
    return _emb_kernel(table, idx)
